# baseline (speedup 1.0000x reference)
import functools

import jax
import jax.numpy as jnp
from jax import lax
from jax.experimental import pallas as pl
from jax.experimental.pallas import tpu as pltpu
from jax.experimental.pallas import tpu_sc as plsc

KNN = 20
EPS = 1e-5
NEG = -3e38
BIG = 3e38
KP = 32


def _topk_body(xn_ref, idx_ref, *, n):
    x = xn_ref[0]
    xb = x.astype(jnp.bfloat16)
    inner = -2.0 * jax.lax.dot_general(xb, xb, (((1,), (1,)), ((), ())),
                                       preferred_element_type=jnp.float32)
    xsq = jnp.sum(x * x, axis=1)
    dist = (xsq[:, None] + inner) + xsq[None, :]
    col = jax.lax.broadcasted_iota(jnp.int32, (n, n), 1).astype(jnp.float32)
    lane = jax.lax.broadcasted_iota(jnp.int32, (n, KP), 1)
    roff = pl.program_id(0) * n

    def step(t, carry):
        g_mat, acc = carry
        rmin = jnp.min(g_mat, axis=1, keepdims=True)
        cand = jnp.where(g_mat == rmin, col, BIG)
        amin = jnp.min(cand, axis=1, keepdims=True)
        onehot = col == amin
        g_mat = jnp.where(onehot, BIG, g_mat)
        gidx = amin.astype(jnp.int32) + roff
        acc = acc + jnp.where(lane == t, gidx, 0)
        return g_mat, acc

    _, acc = jax.lax.fori_loop(0, KNN, step,
                               (dist, jnp.zeros((n, KP), jnp.int32)))
    idx_ref[0] = acc


def _edge_topk(xn):
    b, n, d = xn.shape
    return pl.pallas_call(
        functools.partial(_topk_body, n=n),
        grid=(b,),
        in_specs=[pl.BlockSpec((1, n, d), lambda i: (i, 0, 0))],
        out_specs=pl.BlockSpec((1, n, KP), lambda i: (i, 0, 0)),
        out_shape=jax.ShapeDtypeStruct((b, n, KP), jnp.int32),
    )(xn)


def _sc_gather(table, idx):
    r, d = table.shape
    m = idx.shape[0]
    info = plsc.get_sparse_core_info()
    nc, ns = info.num_cores, info.num_subcores
    nw = nc * ns
    ch = 128
    m_per_w = m // nw
    nch = m_per_w // ch
    mesh = plsc.VectorSubcoreMesh(core_axis_name="c", subcore_axis_name="s")

    @functools.partial(
        pl.kernel, mesh=mesh,
        out_type=jax.ShapeDtypeStruct((m, d), jnp.float32),
        scratch_types=[
            pltpu.VMEM((ch,), jnp.int32),
            pltpu.VMEM((ch, d), jnp.float32),
            pltpu.SemaphoreType.DMA,
        ],
    )
    def k(table_hbm, idx_hbm, out_hbm, idx_v, rows_v, sem):
        wid = lax.axis_index("s") * nc + lax.axis_index("c")
        base = wid * m_per_w

        def body(c, carry):
            off = pl.multiple_of(base + c * ch, ch)
            pltpu.sync_copy(idx_hbm.at[pl.ds(off, ch)], idx_v)
            pltpu.async_copy(table_hbm.at[idx_v], rows_v, sem).wait()
            pltpu.sync_copy(rows_v, out_hbm.at[pl.ds(off, ch)])
            return carry

        jax.lax.fori_loop(0, nch, body, 0)

    return k(table, idx)


def _conv_body(xn_ref, g_ref, w_ref, mx_ref, s1_ref, s2_ref, *, n, din):
    x = xn_ref[0]
    w16 = w_ref[...]
    o = w16.shape[1]
    xc = x[:, :din]
    pad2 = w_ref.shape[0] - 2 * din

    mxa = jnp.full((n, o), NEG, jnp.float32)
    sa = jnp.zeros((n, o), jnp.float32)
    qa = jnp.zeros((n, o), jnp.float32)
    for t in range(KNN):
        xg = g_ref[t, 0]
        parts = [xc, xg[:, :din] - xc]
        if pad2:
            parts.append(jnp.zeros((n, pad2), jnp.float32))
        ef16 = jnp.concatenate(parts, axis=1).astype(jnp.bfloat16)
        z = jnp.dot(ef16, w16, preferred_element_type=jnp.float32)
        mxa = jnp.maximum(mxa, z)
        sa = sa + z
        qa = qa + z * z
    mx_ref[0] = mxa
    s1_ref[0, 0] = jnp.sum(sa, axis=0)
    s2_ref[0, 0] = jnp.sum(qa, axis=0)


def _edge_conv(xn, gath, w16, din):
    b, n, d = xn.shape
    tw, o = w16.shape
    return pl.pallas_call(
        functools.partial(_conv_body, n=n, din=din),
        grid=(b,),
        in_specs=[pl.BlockSpec((1, n, d), lambda i: (i, 0, 0)),
                  pl.BlockSpec((KNN, 1, n, d), lambda i: (0, i, 0, 0)),
                  pl.BlockSpec((tw, o), lambda i: (0, 0))],
        out_specs=[pl.BlockSpec((1, n, o), lambda i: (i, 0, 0)),
                   pl.BlockSpec((1, 1, o), lambda i: (i, 0, 0)),
                   pl.BlockSpec((1, 1, o), lambda i: (i, 0, 0))],
        out_shape=[jax.ShapeDtypeStruct((b, n, o), jnp.float32),
                   jax.ShapeDtypeStruct((b, 1, o), jnp.float32),
                   jax.ShapeDtypeStruct((b, 1, o), jnp.float32)],
    )(xn, gath, w16)


def _finalize_body(mx_ref, s1_ref, s2_ref, g_ref, b_ref, out_ref, *, cnt):
    s1 = jnp.sum(s1_ref[...], axis=(0, 1))
    s2 = jnp.sum(s2_ref[...], axis=(0, 1))
    mean = s1 / cnt
    var = s2 / cnt - mean * mean
    sd = jnp.sqrt(var + EPS)
    u = (mx_ref[...] - mean[None, None, :]) / sd[None, None, :]
    v = u * g_ref[0, 0][None, None, :] + b_ref[0, 0][None, None, :]
    out_ref[...] = jnp.where(v >= 0, v, 0.2 * v)


def _finalize(mx, s1, s2, g, b, cnt):
    return pl.pallas_call(
        functools.partial(_finalize_body, cnt=float(cnt)),
        out_shape=jax.ShapeDtypeStruct(mx.shape, jnp.float32),
    )(mx, s1, s2, g.reshape(1, 1, -1), b.reshape(1, 1, -1))


def _layer5_body(cat_ref, w_ref, mx_ref, s1_ref, s2_ref):
    z = jnp.dot(cat_ref[0].astype(jnp.bfloat16), w_ref[...],
                preferred_element_type=jnp.float32)
    mx_ref[0, 0] = jnp.max(z, axis=0)
    s1_ref[0, 0] = jnp.sum(z, axis=0)
    s2_ref[0, 0] = jnp.sum(z * z, axis=0)


def _layer5(cat, w16):
    b, n, d = cat.shape
    o = w16.shape[1]
    return pl.pallas_call(
        _layer5_body,
        grid=(b,),
        in_specs=[pl.BlockSpec((1, n, d), lambda i: (i, 0, 0)),
                  pl.BlockSpec((d, o), lambda i: (0, 0))],
        out_specs=[pl.BlockSpec((1, 1, o), lambda i: (i, 0, 0)),
                   pl.BlockSpec((1, 1, o), lambda i: (i, 0, 0)),
                   pl.BlockSpec((1, 1, o), lambda i: (i, 0, 0))],
        out_shape=[jax.ShapeDtypeStruct((b, 1, o), jnp.float32),
                   jax.ShapeDtypeStruct((b, 1, o), jnp.float32),
                   jax.ShapeDtypeStruct((b, 1, o), jnp.float32)],
    )(cat, w16)


def kernel(x, W1, g1, b1, W2, g2, b2, W3, g3, b3, W4, g4, b4, W5, g5, b5):
    b, _, n = x.shape
    xt = jnp.swapaxes(x, 2, 1)
    xt = jnp.pad(xt, ((0, 0), (0, 0), (0, 125)))

    feats = []
    cur = xt
    for (w, g, bb) in ((W1, g1, b1), (W2, g2, b2), (W3, g3, b3), (W4, g4, b4)):
        din = w.shape[1] // 2
        wc, wn = w[:, :din], w[:, din:]
        w2 = jnp.concatenate([wc.T, wn.T], axis=0)
        w2 = jnp.pad(w2, ((0, 256 - 2 * din), (0, 0)))
        if cur.shape[2] < 128:
            cur = jnp.pad(cur, ((0, 0), (0, 0), (0, 128 - cur.shape[2])))
        d = cur.shape[2]
        idx = _edge_topk(cur)
        idx_flat = jnp.transpose(idx[:, :, :KNN], (2, 0, 1)).reshape(-1)
        gath = _sc_gather(cur.reshape(b * n, d), idx_flat)
        gath = gath.reshape(KNN, b, n, d)
        mx, s1, s2 = _edge_conv(cur, gath, w2.astype(jnp.bfloat16), din)
        cur = _finalize(mx, s1, s2, g, bb, b * n * KNN)
        feats.append(cur)

    cat = jnp.concatenate(feats, axis=2)
    mxn, s1, s2 = _layer5(cat, W5.T.astype(jnp.bfloat16))
    out = _finalize(mxn, s1, s2, g5, b5, b * n)
    return out.reshape(b, -1)


# speedup vs baseline: 1.1212x; 1.1212x over previous
import functools

import jax
import jax.numpy as jnp
from jax import lax
from jax.experimental import pallas as pl
from jax.experimental.pallas import tpu as pltpu
from jax.experimental.pallas import tpu_sc as plsc

KNN = 20
EPS = 1e-5
NEG = -3e38
BIG = 3e38
KP = 32


def _topk_body(xn_ref, idx_ref, *, n):
    x = xn_ref[0]
    xb = x.astype(jnp.bfloat16)
    inner = -2.0 * jax.lax.dot_general(xb, xb, (((1,), (1,)), ((), ())),
                                       preferred_element_type=jnp.float32)
    xsq = jnp.sum(x * x, axis=1)
    dist = (xsq[:, None] + inner) + xsq[None, :]
    col = jax.lax.broadcasted_iota(jnp.int32, (n, n), 1).astype(jnp.float32)
    lane = jax.lax.broadcasted_iota(jnp.int32, (n, KP), 1)
    roff = pl.program_id(0) * n

    def step(t, carry):
        g_mat, acc = carry
        rmin = jnp.min(g_mat, axis=1, keepdims=True)
        cand = jnp.where(g_mat == rmin, col, BIG)
        amin = jnp.min(cand, axis=1, keepdims=True)
        onehot = col == amin
        g_mat = jnp.where(onehot, BIG, g_mat)
        gidx = amin.astype(jnp.int32) + roff
        acc = acc + jnp.where(lane == t, gidx, 0)
        return g_mat, acc

    _, acc = jax.lax.fori_loop(0, KNN, step,
                               (dist, jnp.zeros((n, KP), jnp.int32)))
    idx_ref[0] = acc


def _edge_topk(xn):
    b, n, d = xn.shape
    return pl.pallas_call(
        functools.partial(_topk_body, n=n),
        grid=(b,),
        in_specs=[pl.BlockSpec((1, n, d), lambda i: (i, 0, 0))],
        out_specs=pl.BlockSpec((1, n, KP), lambda i: (i, 0, 0)),
        out_shape=jax.ShapeDtypeStruct((b, n, KP), jnp.int32),
    )(xn)


def _sc_gather(table, idx):
    r, d = table.shape
    m = idx.shape[0]
    info = plsc.get_sparse_core_info()
    nc, ns = info.num_cores, info.num_subcores
    nw = nc * ns
    ch = 128
    m_per_w = m // nw
    nch = m_per_w // ch
    mesh = plsc.VectorSubcoreMesh(core_axis_name="c", subcore_axis_name="s")

    @functools.partial(
        pl.kernel, mesh=mesh,
        out_type=jax.ShapeDtypeStruct((m, d), jnp.float32),
        scratch_types=[
            pltpu.VMEM((nch, ch), jnp.int32),
            pltpu.VMEM((2, ch, d), jnp.float32),
            pltpu.SemaphoreType.DMA,
            pltpu.SemaphoreType.DMA,
        ],
    )
    def k(table_hbm, idx_hbm, out_hbm, idx_v, rows_v, sem_a, sem_b):
        wid = lax.axis_index("s") * nc + lax.axis_index("c")
        base = wid * m_per_w
        pltpu.sync_copy(idx_hbm.at[pl.ds(wid * nch, nch)], idx_v)
        pltpu.async_copy(table_hbm.at[idx_v.at[0]], rows_v.at[0], sem_a)

        def body(c2, carry):
            c0 = c2 * 2
            pltpu.async_copy(table_hbm.at[idx_v.at[c0 + 1]], rows_v.at[1],
                             sem_b)
            pltpu.make_async_copy(table_hbm.at[idx_v.at[c0]], rows_v.at[0],
                                  sem_a).wait()
            pltpu.sync_copy(rows_v.at[0], out_hbm.at[pl.ds(base + c0 * ch, ch)])

            @pl.when(c0 + 2 < nch)
            def _():
                pltpu.async_copy(table_hbm.at[idx_v.at[c0 + 2]], rows_v.at[0],
                                 sem_a)

            pltpu.make_async_copy(table_hbm.at[idx_v.at[c0 + 1]], rows_v.at[1],
                                  sem_b).wait()
            pltpu.sync_copy(rows_v.at[1],
                            out_hbm.at[pl.ds(base + (c0 + 1) * ch, ch)])
            return carry

        jax.lax.fori_loop(0, nch // 2, body, 0)

    return k(table, idx.reshape(nw * nch, ch))


def _conv_body(xn_ref, g_ref, w_ref, mx_ref, s1_ref, s2_ref, *, n, din):
    x = xn_ref[0]
    w16 = w_ref[...]
    o = w16.shape[1]
    xc = x[:, :din]
    pad2 = w_ref.shape[0] - 2 * din

    mxa = jnp.full((n, o), NEG, jnp.float32)
    sa = jnp.zeros((n, o), jnp.float32)
    qa = jnp.zeros((n, o), jnp.float32)
    for t in range(KNN):
        xg = g_ref[t, 0]
        parts = [xc, xg[:, :din] - xc]
        if pad2:
            parts.append(jnp.zeros((n, pad2), jnp.float32))
        ef16 = jnp.concatenate(parts, axis=1).astype(jnp.bfloat16)
        z = jnp.dot(ef16, w16, preferred_element_type=jnp.float32)
        mxa = jnp.maximum(mxa, z)
        sa = sa + z
        qa = qa + z * z
    mx_ref[0] = mxa
    s1_ref[0, 0] = jnp.sum(sa, axis=0)
    s2_ref[0, 0] = jnp.sum(qa, axis=0)


def _edge_conv(xn, gath, w16, din):
    b, n, d = xn.shape
    tw, o = w16.shape
    return pl.pallas_call(
        functools.partial(_conv_body, n=n, din=din),
        grid=(b,),
        in_specs=[pl.BlockSpec((1, n, d), lambda i: (i, 0, 0)),
                  pl.BlockSpec((KNN, 1, n, d), lambda i: (0, i, 0, 0)),
                  pl.BlockSpec((tw, o), lambda i: (0, 0))],
        out_specs=[pl.BlockSpec((1, n, o), lambda i: (i, 0, 0)),
                   pl.BlockSpec((1, 1, o), lambda i: (i, 0, 0)),
                   pl.BlockSpec((1, 1, o), lambda i: (i, 0, 0))],
        out_shape=[jax.ShapeDtypeStruct((b, n, o), jnp.float32),
                   jax.ShapeDtypeStruct((b, 1, o), jnp.float32),
                   jax.ShapeDtypeStruct((b, 1, o), jnp.float32)],
    )(xn, gath, w16)


def _finalize_body(mx_ref, s1_ref, s2_ref, g_ref, b_ref, out_ref, *, cnt):
    s1 = jnp.sum(s1_ref[...], axis=(0, 1))
    s2 = jnp.sum(s2_ref[...], axis=(0, 1))
    mean = s1 / cnt
    var = s2 / cnt - mean * mean
    sd = jnp.sqrt(var + EPS)
    u = (mx_ref[...] - mean[None, None, :]) / sd[None, None, :]
    v = u * g_ref[0, 0][None, None, :] + b_ref[0, 0][None, None, :]
    out_ref[...] = jnp.where(v >= 0, v, 0.2 * v)


def _finalize(mx, s1, s2, g, b, cnt):
    return pl.pallas_call(
        functools.partial(_finalize_body, cnt=float(cnt)),
        out_shape=jax.ShapeDtypeStruct(mx.shape, jnp.float32),
    )(mx, s1, s2, g.reshape(1, 1, -1), b.reshape(1, 1, -1))


def _layer5_body(cat_ref, w_ref, mx_ref, s1_ref, s2_ref):
    z = jnp.dot(cat_ref[0].astype(jnp.bfloat16), w_ref[...],
                preferred_element_type=jnp.float32)
    mx_ref[0, 0] = jnp.max(z, axis=0)
    s1_ref[0, 0] = jnp.sum(z, axis=0)
    s2_ref[0, 0] = jnp.sum(z * z, axis=0)


def _layer5(cat, w16):
    b, n, d = cat.shape
    o = w16.shape[1]
    return pl.pallas_call(
        _layer5_body,
        grid=(b,),
        in_specs=[pl.BlockSpec((1, n, d), lambda i: (i, 0, 0)),
                  pl.BlockSpec((d, o), lambda i: (0, 0))],
        out_specs=[pl.BlockSpec((1, 1, o), lambda i: (i, 0, 0)),
                   pl.BlockSpec((1, 1, o), lambda i: (i, 0, 0)),
                   pl.BlockSpec((1, 1, o), lambda i: (i, 0, 0))],
        out_shape=[jax.ShapeDtypeStruct((b, 1, o), jnp.float32),
                   jax.ShapeDtypeStruct((b, 1, o), jnp.float32),
                   jax.ShapeDtypeStruct((b, 1, o), jnp.float32)],
    )(cat, w16)


def kernel(x, W1, g1, b1, W2, g2, b2, W3, g3, b3, W4, g4, b4, W5, g5, b5):
    b, _, n = x.shape
    xt = jnp.swapaxes(x, 2, 1)
    xt = jnp.pad(xt, ((0, 0), (0, 0), (0, 125)))

    feats = []
    cur = xt
    for (w, g, bb) in ((W1, g1, b1), (W2, g2, b2), (W3, g3, b3), (W4, g4, b4)):
        din = w.shape[1] // 2
        wc, wn = w[:, :din], w[:, din:]
        w2 = jnp.concatenate([wc.T, wn.T], axis=0)
        w2 = jnp.pad(w2, ((0, 256 - 2 * din), (0, 0)))
        if cur.shape[2] < 128:
            cur = jnp.pad(cur, ((0, 0), (0, 0), (0, 128 - cur.shape[2])))
        d = cur.shape[2]
        idx = _edge_topk(cur)
        idx_flat = jnp.transpose(idx[:, :, :KNN], (2, 0, 1)).reshape(-1)
        gath = _sc_gather(cur.reshape(b * n, d), idx_flat)
        gath = gath.reshape(KNN, b, n, d)
        mx, s1, s2 = _edge_conv(cur, gath, w2.astype(jnp.bfloat16), din)
        cur = _finalize(mx, s1, s2, g, bb, b * n * KNN)
        feats.append(cur)

    cat = jnp.concatenate(feats, axis=2)
    mxn, s1, s2 = _layer5(cat, W5.T.astype(jnp.bfloat16))
    out = _finalize(mxn, s1, s2, g5, b5, b * n)
    return out.reshape(b, -1)
